# baseline (speedup 1.0000x reference)
import functools

import jax
import jax.numpy as jnp
from jax import lax
from jax.experimental import pallas as pl
from jax.experimental.pallas import tpu as pltpu
from jax.experimental.pallas import tpu_sc as plsc


def kernel(x, edge_index):
    n_nodes, d = x.shape
    b = edge_index.shape[1]
    src = edge_index[0].astype(jnp.int32)

    info = plsc.get_sparse_core_info()
    nc, ns = info.num_cores, info.num_subcores
    nw = nc * ns
    b_per_w = b // nw
    chunk = 80
    n_groups = b_per_w // chunk
    m = 4

    mesh = plsc.VectorSubcoreMesh(core_axis_name="c", subcore_axis_name="s")

    @functools.partial(
        pl.kernel,
        mesh=mesh,
        out_type=jax.ShapeDtypeStruct((b, d), x.dtype),
        scratch_types=[
            pltpu.VMEM((b_per_w,), jnp.int32),
            [pltpu.VMEM((chunk, d), jnp.float32) for _ in range(m)],
            pltpu.VMEM_SHARED((n_nodes, d), jnp.float32),
            pltpu.SemaphoreType.DMA,
            pltpu.SemaphoreType.DMA,
            [pltpu.SemaphoreType.DMA for _ in range(m)],
        ],
    )
    def gather_kernel(x_hbm, ei_hbm, out_hbm, idx_v, bufs, x_s, gsem, hsem,
                      wsems):
        sid = lax.axis_index("s")
        wid = sid * nc + lax.axis_index("c")
        base = wid * b_per_w

        rows_per_tile = (n_nodes // ns) // 8 * 8
        tail = n_nodes - ns * rows_per_tile
        pltpu.sync_copy(x_hbm.at[pl.ds(sid * rows_per_tile, rows_per_tile)],
                        x_s.at[pl.ds(sid * rows_per_tile, rows_per_tile)])

        @pl.when(sid == ns - 1)
        def _copy_tail():
            pltpu.sync_copy(x_hbm.at[pl.ds(ns * rows_per_tile, tail)],
                            x_s.at[pl.ds(ns * rows_per_tile, tail)])

        pltpu.sync_copy(ei_hbm.at[pl.ds(base, b_per_w)], idx_v)

        def fire_g(g, o, from_hbm=False):
            src_ref = x_hbm if from_hbm else x_s
            pltpu.async_copy(
                src_ref.at[idx_v.at[pl.ds(g * chunk, chunk)]], bufs[o],
                hsem if from_hbm else gsem)

        def wait_g(o, from_hbm=False):
            src_ref = x_hbm if from_hbm else x_s
            pltpu.make_async_copy(
                src_ref.at[idx_v.at[pl.ds(0, chunk)]], bufs[o],
                hsem if from_hbm else gsem).wait()

        def fire_w(g, o):
            pltpu.async_copy(
                bufs[o], out_hbm.at[pl.ds(base + g * chunk, chunk)], wsems[o])

        def wait_w(g, o):
            pltpu.make_async_copy(
                bufs[o], out_hbm.at[pl.ds(base + g * chunk, chunk)],
                wsems[o]).wait()

        def slot(h, o, first=False, prefetch=True, pf_hbm=False,
                 wait_hbm=False):
            wait_g(o, from_hbm=wait_hbm)
            if prefetch:
                fire_g(h + 2, (o + 2) % m, from_hbm=pf_hbm)
            fire_w(h, o)
            if not first:
                wait_w(h - 1, (o + m - 1) % m)

        fire_g(0, 0, from_hbm=True)
        fire_g(1, 1, from_hbm=True)
        plsc.subcore_barrier()
        for h in range(m):
            slot(h, h, first=(h == 0), pf_hbm=(h < 2), wait_hbm=(h < 4))

        def body(t, carry):
            h = m * t
            for o in range(m):
                slot(h + o, o)
            return carry

        t_hi = (n_groups - 2) // m
        lax.fori_loop(1, t_hi, body, 0)

        for h in range(m * t_hi, n_groups):
            slot(h, h % m, prefetch=(h + 2 < n_groups), pf_hbm=True,
                 wait_hbm=(h >= m * t_hi + 2))

        wait_w(n_groups - 1, (n_groups - 1) % m)

    return gather_kernel(x, src)


# speedup vs baseline: 1.0224x; 1.0224x over previous
import functools

import jax
import jax.numpy as jnp
from jax import lax
from jax.experimental import pallas as pl
from jax.experimental.pallas import tpu as pltpu
from jax.experimental.pallas import tpu_sc as plsc


def kernel(x, edge_index):
    n_nodes, d = x.shape
    b = edge_index.shape[1]
    src = edge_index[0].astype(jnp.int32)

    info = plsc.get_sparse_core_info()
    nc, ns = info.num_cores, info.num_subcores
    nw = nc * ns
    b_per_w = b // nw
    chunk = 80
    n_groups = b_per_w // chunk
    m = 4

    mesh = plsc.VectorSubcoreMesh(core_axis_name="c", subcore_axis_name="s")

    @functools.partial(
        pl.kernel,
        mesh=mesh,
        out_type=jax.ShapeDtypeStruct((b, d), x.dtype),
        scratch_types=[
            pltpu.VMEM((b_per_w,), jnp.int32),
            [pltpu.VMEM((chunk, d), jnp.float32) for _ in range(m)],
            pltpu.VMEM_SHARED((n_nodes, d), jnp.float32),
            pltpu.SemaphoreType.DMA,
            pltpu.SemaphoreType.DMA,
            [pltpu.SemaphoreType.DMA for _ in range(m)],
        ],
    )
    def gather_kernel(x_hbm, ei_hbm, out_hbm, idx_v, bufs, x_s, gsem, hsem,
                      wsems):
        sid = lax.axis_index("s")
        wid = sid * nc + lax.axis_index("c")
        base = wid * b_per_w

        rows_per_tile = (n_nodes // ns) // 8 * 8
        tail = n_nodes - ns * rows_per_tile
        pltpu.sync_copy(x_hbm.at[pl.ds(sid * rows_per_tile, rows_per_tile)],
                        x_s.at[pl.ds(sid * rows_per_tile, rows_per_tile)])

        @pl.when(sid == ns - 1)
        def _copy_tail():
            pltpu.sync_copy(x_hbm.at[pl.ds(ns * rows_per_tile, tail)],
                            x_s.at[pl.ds(ns * rows_per_tile, tail)])

        pltpu.sync_copy(ei_hbm.at[pl.ds(base, b_per_w)], idx_v)

        def fire_g(g, o, from_hbm=False):
            src_ref = x_hbm if from_hbm else x_s
            pltpu.async_copy(
                src_ref.at[idx_v.at[pl.ds(g * chunk, chunk)]], bufs[o],
                hsem if from_hbm else gsem)

        def wait_g(o, from_hbm=False):
            src_ref = x_hbm if from_hbm else x_s
            pltpu.make_async_copy(
                src_ref.at[idx_v.at[pl.ds(0, chunk)]], bufs[o],
                hsem if from_hbm else gsem).wait()

        def fire_w(g, o):
            pltpu.async_copy(
                bufs[o], out_hbm.at[pl.ds(base + g * chunk, chunk)], wsems[o])

        def wait_w(g, o):
            pltpu.make_async_copy(
                bufs[o], out_hbm.at[pl.ds(base + g * chunk, chunk)],
                wsems[o]).wait()

        def slot(h, o, first=False, prefetch=True, pf_hbm=False,
                 wait_hbm=False):
            wait_g(o, from_hbm=wait_hbm)
            if prefetch:
                fire_g(h + 2, (o + 2) % m, from_hbm=pf_hbm)
            fire_w(h, o)
            if not first:
                wait_w(h - 1, (o + m - 1) % m)

        fire_g(0, 0, from_hbm=True)
        fire_g(1, 1, from_hbm=True)
        plsc.subcore_barrier()
        for h in range(m):
            slot(h, h, first=(h == 0), wait_hbm=(h < 2))

        def body(t, carry):
            h = m * t
            for o in range(m):
                slot(h + o, o)
            return carry

        t_hi = (n_groups - 2) // m
        lax.fori_loop(1, t_hi, body, 0)

        for h in range(m * t_hi, n_groups):
            slot(h, h % m, prefetch=(h + 2 < n_groups))

        wait_w(n_groups - 1, (n_groups - 1) % m)

    return gather_kernel(x, src)


# speedup vs baseline: 1.0304x; 1.0078x over previous
import functools

import jax
import jax.numpy as jnp
from jax import lax
from jax.experimental import pallas as pl
from jax.experimental.pallas import tpu as pltpu
from jax.experimental.pallas import tpu_sc as plsc


def kernel(x, edge_index):
    n_nodes, d = x.shape
    b = edge_index.shape[1]
    src = edge_index[0].astype(jnp.int32)

    info = plsc.get_sparse_core_info()
    nc, ns = info.num_cores, info.num_subcores
    nw = nc * ns
    b_per_w = b // nw
    chunk = 80
    n_groups = b_per_w // chunk
    m = 4

    mesh = plsc.VectorSubcoreMesh(core_axis_name="c", subcore_axis_name="s")

    @functools.partial(
        pl.kernel,
        mesh=mesh,
        out_type=jax.ShapeDtypeStruct((b, d), x.dtype),
        scratch_types=[
            pltpu.VMEM((b_per_w,), jnp.int32),
            [pltpu.VMEM((chunk, d), jnp.float32) for _ in range(m)],
            pltpu.VMEM_SHARED((n_nodes, d), jnp.float32),
            pltpu.SemaphoreType.DMA,
            pltpu.SemaphoreType.DMA,
            [pltpu.SemaphoreType.DMA for _ in range(m)],
        ],
    )
    def gather_kernel(x_hbm, ei_hbm, out_hbm, idx_v, bufs, x_s, gsem, hsem,
                      wsems):
        sid = lax.axis_index("s")
        wid = sid * nc + lax.axis_index("c")
        base = wid * b_per_w

        rows_per_tile = (n_nodes // ns) // 8 * 8
        tail = n_nodes - ns * rows_per_tile
        pltpu.sync_copy(x_hbm.at[pl.ds(sid * rows_per_tile, rows_per_tile)],
                        x_s.at[pl.ds(sid * rows_per_tile, rows_per_tile)])

        @pl.when(sid == ns - 1)
        def _copy_tail():
            pltpu.sync_copy(x_hbm.at[pl.ds(ns * rows_per_tile, tail)],
                            x_s.at[pl.ds(ns * rows_per_tile, tail)])

        pltpu.sync_copy(ei_hbm.at[pl.ds(base, b_per_w)], idx_v)

        def fire_g(g, o, from_hbm=False):
            src_ref = x_hbm if from_hbm else x_s
            pltpu.async_copy(
                src_ref.at[idx_v.at[pl.ds(g * chunk, chunk)]], bufs[o],
                hsem if from_hbm else gsem)

        def wait_g(o, from_hbm=False):
            src_ref = x_hbm if from_hbm else x_s
            pltpu.make_async_copy(
                src_ref.at[idx_v.at[pl.ds(0, chunk)]], bufs[o],
                hsem if from_hbm else gsem).wait()

        def fire_w(g, o):
            pltpu.async_copy(
                bufs[o], out_hbm.at[pl.ds(base + g * chunk, chunk)], wsems[o])

        def wait_w(g, o):
            pltpu.make_async_copy(
                bufs[o], out_hbm.at[pl.ds(base + g * chunk, chunk)],
                wsems[o]).wait()

        def slot(h, o, first=False, prefetch=True, pf_hbm=False,
                 wait_hbm=False):
            wait_g(o, from_hbm=wait_hbm)
            if prefetch:
                fire_g(h + 2, (o + 2) % m, from_hbm=pf_hbm)
            fire_w(h, o)
            if not first:
                wait_w(h - 1, (o + m - 1) % m)

        plsc.subcore_barrier()
        fire_g(0, 0)
        fire_g(1, 1)
        for h in range(m):
            slot(h, h, first=(h == 0))

        def body(t, carry):
            h = m * t
            for o in range(m):
                slot(h + o, o)
            return carry

        t_hi = (n_groups - 2) // m
        lax.fori_loop(1, t_hi, body, 0)

        for h in range(m * t_hi, n_groups):
            slot(h, h % m, prefetch=(h + 2 < n_groups))

        wait_w(n_groups - 1, (n_groups - 1) % m)

    return gather_kernel(x, src)


# speedup vs baseline: 1.0318x; 1.0014x over previous
import functools

import jax
import jax.numpy as jnp
from jax import lax
from jax.experimental import pallas as pl
from jax.experimental.pallas import tpu as pltpu
from jax.experimental.pallas import tpu_sc as plsc


def kernel(x, edge_index):
    n_nodes, d = x.shape
    b = edge_index.shape[1]
    src = edge_index[0].astype(jnp.int32)

    info = plsc.get_sparse_core_info()
    nc, ns = info.num_cores, info.num_subcores
    nw = nc * ns
    b_per_w = b // nw
    chunk = 80
    n_groups = b_per_w // chunk
    m = 4

    mesh = plsc.VectorSubcoreMesh(core_axis_name="c", subcore_axis_name="s")

    @functools.partial(
        pl.kernel,
        mesh=mesh,
        out_type=jax.ShapeDtypeStruct((b, d), x.dtype),
        scratch_types=[
            pltpu.VMEM((b_per_w,), jnp.int32),
            [pltpu.VMEM((chunk, d), jnp.float32) for _ in range(m)],
            pltpu.VMEM_SHARED((n_nodes, d), jnp.float32),
            pltpu.SemaphoreType.DMA,
            pltpu.SemaphoreType.DMA,
            [pltpu.SemaphoreType.DMA for _ in range(m)],
        ],
    )
    def gather_kernel(x_hbm, ei_hbm, out_hbm, idx_v, bufs, x_s, gsem, hsem,
                      wsems):
        sid = lax.axis_index("s")
        wid = sid * nc + lax.axis_index("c")
        base = wid * b_per_w

        rows_per_tile = (n_nodes // ns) // 8 * 8
        tail = n_nodes - ns * rows_per_tile
        pltpu.sync_copy(x_hbm.at[pl.ds(sid * rows_per_tile, rows_per_tile)],
                        x_s.at[pl.ds(sid * rows_per_tile, rows_per_tile)])

        @pl.when(sid == ns - 1)
        def _copy_tail():
            pltpu.sync_copy(x_hbm.at[pl.ds(ns * rows_per_tile, tail)],
                            x_s.at[pl.ds(ns * rows_per_tile, tail)])

        pltpu.sync_copy(ei_hbm.at[pl.ds(base, b_per_w)], idx_v)

        def fire_g(g, o, from_hbm=False):
            src_ref = x_hbm if from_hbm else x_s
            pltpu.async_copy(
                src_ref.at[idx_v.at[pl.ds(g * chunk, chunk)]], bufs[o],
                hsem if from_hbm else gsem)

        def wait_g(o, from_hbm=False):
            src_ref = x_hbm if from_hbm else x_s
            pltpu.make_async_copy(
                src_ref.at[idx_v.at[pl.ds(0, chunk)]], bufs[o],
                hsem if from_hbm else gsem).wait()

        def fire_w(g, o):
            pltpu.async_copy(
                bufs[o], out_hbm.at[pl.ds(base + g * chunk, chunk)], wsems[o])

        def wait_w(g, o):
            pltpu.make_async_copy(
                bufs[o], out_hbm.at[pl.ds(base + g * chunk, chunk)],
                wsems[o]).wait()

        def slot(h, o, first=False, prefetch=True):
            wait_g(o)
            fire_w(h, o)
            if not first:
                wait_w(h - 1, (o + m - 1) % m)
            if prefetch:
                fire_g(h + 3, (o + 3) % m)

        plsc.subcore_barrier()
        fire_g(0, 0)
        fire_g(1, 1)
        fire_g(2, 2)
        for h in range(m):
            slot(h, h, first=(h == 0))

        def body(t, carry):
            h = m * t
            for o in range(m):
                slot(h + o, o)
            return carry

        t_hi = (n_groups - 3) // m
        lax.fori_loop(1, t_hi, body, 0)

        for h in range(m * t_hi, n_groups):
            slot(h, h % m, prefetch=(h + 3 < n_groups))

        wait_w(n_groups - 1, (n_groups - 1) % m)

    return gather_kernel(x, src)
